# baseline (speedup 1.0000x reference)
import jax
import jax.numpy as jnp
from jax import lax
from jax.experimental import pallas as pl
from jax.experimental.pallas import tpu as pltpu
from jax.experimental.pallas import tpu_sc as plsc

VOCAB = 1000000
DIM = 16
B, L = 16384, 200
N = B * L
NC, NS, LANES = 2, 16, 16
NW = NC * NS
SB = 1024
NSB = B // SB
UNITS = L * NSB
UPT = UNITS // NW
BT = B // 128
HALF = 8 * 8 * 128


def _body(idx_hbm, table_hbm, out_hbm, idxb, rows0, rows1, stag0, stag1,
          gsem, wsem):
    wid = lax.axis_index("s") * NC + lax.axis_index("c")
    u0 = wid * UPT
    rows = (rows0, rows1)
    stag = (stag0, stag1)
    riota = lax.iota(jnp.int32, LANES)
    cols = [(riota + s) & (DIM - 1) for s in range(DIM)]
    woffs = [(cols[s] >> 3) * HALF + (cols[s] & 7) * 128 + riota
             for s in range(DIM)]

    def unit_lsb(t):
        u = u0 + t
        return u >> 4, u & (NSB - 1)

    def load_add(t, buf):
        l, sb = unit_lsb(t)
        off = l * B + sb * SB
        pltpu.sync_copy(idx_hbm.at[pl.ds(off, SB)], idxb.at[buf])

        def add_one(i, carry):
            sl = pl.ds(i * LANES, LANES)
            idxb[buf, sl] = idxb[buf, sl] + 1
            return carry

        lax.fori_loop(0, SB // LANES, add_one, 0, unroll=8)

    def start_gather(buf):
        pltpu.async_copy(table_hbm.at[idxb.at[buf]], rows[buf], gsem)

    def wait_gather(buf):
        pltpu.make_async_copy(table_hbm.at[idxb.at[buf]], rows[buf],
                              gsem).wait()

    def transpose(buf):
        rbuf = rows[buf]
        sbuf = stag[buf]

        def step(bt8, carry):
            for kb in range(8):
                i0 = bt8 * 128 + kb * 16
                base = bt8 * 1024 + kb * 16
                row_idx = riota + i0
                vs = [plsc.load_gather(rbuf, [row_idx, cols[s]])
                      for s in range(DIM)]
                for s in range(DIM):
                    plsc.store_scatter(sbuf, [woffs[s] + base], vs[s])
            return carry

        lax.fori_loop(0, 8, step, 0)

    def start_write(t, buf):
        l, sb = unit_lsb(t)
        for dt in (0, 1):
            pltpu.async_copy(stag[buf].at[pl.ds(dt * HALF, HALF)],
                             out_hbm.at[l, dt, pl.ds(sb * HALF, HALF)], wsem)

    def wait_write(t, buf):
        l, sb = unit_lsb(t)
        for dt in (0, 1):
            pltpu.make_async_copy(stag[buf].at[pl.ds(dt * HALF, HALF)],
                                  out_hbm.at[l, dt, pl.ds(sb * HALF, HALF)],
                                  wsem).wait()

    load_add(0, 0)
    start_gather(0)

    def outer(tt, carry):
        for b in (0, 1):
            t = tt * 2 + b
            bnext = 1 - b

            @pl.when(t + 1 < UPT)
            def _():
                load_add(t + 1, bnext)

            wait_gather(b)

            @pl.when(t + 1 < UPT)
            def _():
                start_gather(bnext)

            @pl.when(t >= 2)
            def _():
                wait_write(t - 2, b)

            start_write(t, b)
        return carry

    lax.fori_loop(0, UPT // 2, outer, 0)
    wait_write(UPT - 2, 0)
    wait_write(UPT - 1, 1)


def kernel(action, table):
    idx1d = action.reshape(B, L).T.reshape(N)
    mesh = plsc.VectorSubcoreMesh(
        core_axis_name="c", subcore_axis_name="s", num_cores=NC,
        num_subcores=NS)
    out3 = pl.kernel(
        _body,
        out_type=jax.ShapeDtypeStruct((L, 2, BT * 8 * 128), jnp.float32),
        mesh=mesh,
        scratch_types=[
            pltpu.VMEM((2, SB), jnp.int32),
            pltpu.VMEM((SB, DIM), jnp.float32),
            pltpu.VMEM((SB, DIM), jnp.float32),
            pltpu.VMEM((2 * HALF,), jnp.float32),
            pltpu.VMEM((2 * HALF,), jnp.float32),
            pltpu.SemaphoreType.DMA,
            pltpu.SemaphoreType.DMA,
        ],
        compiler_params=pltpu.CompilerParams(use_tc_tiling_on_sc=False,
                                             needs_layout_passes=False),
    )(idx1d, table)
    out6 = out3.reshape(L, 2, BT, 8, 128)
    return out6.transpose(2, 4, 0, 1, 3).reshape(B, L, DIM)


# speedup vs baseline: 1.0460x; 1.0460x over previous
import jax
import jax.numpy as jnp
from jax import lax
from jax.experimental import pallas as pl
from jax.experimental.pallas import tpu as pltpu
from jax.experimental.pallas import tpu_sc as plsc

VOCAB = 1000000
DIM = 16
B, L = 16384, 200
N = B * L
NC, NS, LANES = 2, 16, 16
NW = NC * NS
SB = 1024
NSB = B // SB
UNITS = L * NSB
UPT = UNITS // NW
BT = B // 128
HALF = 8 * 8 * 128


def _body(idx_hbm, table_hbm, out_hbm, idxb, rows0, rows1, stag0, stag1,
          gsem, wsem):
    wid = lax.axis_index("s") * NC + lax.axis_index("c")
    u0 = wid * UPT
    rows = (rows0, rows1)
    stag = (stag0, stag1)
    riota = lax.iota(jnp.int32, LANES)
    cols = [(riota + s) & (DIM - 1) for s in range(DIM)]
    woffs = [(cols[s] >> 3) * HALF + (cols[s] & 7) * 128 + riota
             for s in range(DIM)]

    def unit_lsb(t):
        u = u0 + t
        return u >> 4, u & (NSB - 1)

    def load_add(t, buf):
        l, sb = unit_lsb(t)
        off = l * B + sb * SB
        pltpu.sync_copy(idx_hbm.at[pl.ds(off, SB)], idxb.at[buf])

        def add_one(i, carry):
            sl = pl.ds(i * LANES, LANES)
            idxb[buf, sl] = idxb[buf, sl] + 1
            return carry

        lax.fori_loop(0, SB // LANES, add_one, 0, unroll=8)

    GH = SB // 2

    def start_gather(buf):
        for h in (0, 1):
            pltpu.async_copy(table_hbm.at[idxb.at[buf, pl.ds(h * GH, GH)]],
                             rows[buf].at[pl.ds(h * GH, GH)], gsem)

    def wait_gather(buf):
        for h in (0, 1):
            pltpu.make_async_copy(
                table_hbm.at[idxb.at[buf, pl.ds(h * GH, GH)]],
                rows[buf].at[pl.ds(h * GH, GH)], gsem).wait()

    def transpose(buf):
        rbuf = rows[buf]
        sbuf = stag[buf]

        def step(bt8, carry):
            for kb in range(8):
                i0 = bt8 * 128 + kb * 16
                base = bt8 * 1024 + kb * 16
                row_idx = riota + i0
                vs = [plsc.load_gather(rbuf, [row_idx, cols[s]])
                      for s in range(DIM)]
                for s in range(DIM):
                    plsc.store_scatter(sbuf, [woffs[s] + base], vs[s])
            return carry

        lax.fori_loop(0, 8, step, 0)

    def start_write(t, buf):
        l, sb = unit_lsb(t)
        for dt in (0, 1):
            pltpu.async_copy(stag[buf].at[pl.ds(dt * HALF, HALF)],
                             out_hbm.at[l, dt, pl.ds(sb * HALF, HALF)], wsem)

    def wait_write(t, buf):
        l, sb = unit_lsb(t)
        for dt in (0, 1):
            pltpu.make_async_copy(stag[buf].at[pl.ds(dt * HALF, HALF)],
                                  out_hbm.at[l, dt, pl.ds(sb * HALF, HALF)],
                                  wsem).wait()

    load_add(0, 0)
    start_gather(0)

    def outer(tt, carry):
        for b in (0, 1):
            t = tt * 2 + b
            bnext = 1 - b

            @pl.when(t + 1 < UPT)
            def _():
                load_add(t + 1, bnext)

            wait_gather(b)

            @pl.when(t + 1 < UPT)
            def _():
                start_gather(bnext)

            @pl.when(t >= 2)
            def _():
                wait_write(t - 2, b)

            transpose(b)
            start_write(t, b)
        return carry

    lax.fori_loop(0, UPT // 2, outer, 0)
    wait_write(UPT - 2, 0)
    wait_write(UPT - 1, 1)


def kernel(action, table):
    idx1d = action.reshape(B, L).T.reshape(N)
    mesh = plsc.VectorSubcoreMesh(
        core_axis_name="c", subcore_axis_name="s", num_cores=NC,
        num_subcores=NS)
    out3 = pl.kernel(
        _body,
        out_type=jax.ShapeDtypeStruct((L, 2, BT * 8 * 128), jnp.float32),
        mesh=mesh,
        scratch_types=[
            pltpu.VMEM((2, SB), jnp.int32),
            pltpu.VMEM((SB, DIM), jnp.float32),
            pltpu.VMEM((SB, DIM), jnp.float32),
            pltpu.VMEM((2 * HALF,), jnp.float32),
            pltpu.VMEM((2 * HALF,), jnp.float32),
            pltpu.SemaphoreType.DMA,
            pltpu.SemaphoreType.DMA,
        ],
        compiler_params=pltpu.CompilerParams(use_tc_tiling_on_sc=False,
                                             needs_layout_passes=False),
    )(idx1d, table)
    out6 = out3.reshape(L, 2, BT, 8, 128)
    return out6.transpose(2, 4, 0, 1, 3).reshape(B, L, DIM)


# speedup vs baseline: 1.0650x; 1.0181x over previous
import jax
import jax.numpy as jnp
from jax import lax
from jax.experimental import pallas as pl
from jax.experimental.pallas import tpu as pltpu
from jax.experimental.pallas import tpu_sc as plsc

VOCAB = 1000000
DIM = 16
B, L = 16384, 200
N = B * L
NC, NS, LANES = 2, 16, 16
NW = NC * NS
SB = 1024
NSB = B // SB
UNITS = L * NSB
UPT = UNITS // NW
BT = B // 128
HALF = 8 * 8 * 128


def _body(idx_hbm, table_hbm, out_hbm, idxb, rows0, rows1, stag0, stag1,
          gsem, wsem):
    wid = lax.axis_index("s") * NC + lax.axis_index("c")
    u0 = wid * UPT
    rows = (rows0, rows1)
    stag = (stag0, stag1)
    riota = lax.iota(jnp.int32, LANES)
    cols = [(riota + s) & (DIM - 1) for s in range(DIM)]
    woffs = [(cols[s] >> 3) * HALF + (cols[s] & 7) * 128 + riota
             for s in range(DIM)]

    def unit_lsb(t):
        u = u0 + t
        return u >> 4, u & (NSB - 1)

    def load_add(t, buf):
        l, sb = unit_lsb(t)
        pltpu.sync_copy(idx_hbm.at[l, pl.ds(sb * SB, SB)], idxb.at[buf])

    def start_gather(buf):
        pltpu.async_copy(table_hbm.at[idxb.at[buf]], rows[buf], gsem)

    def wait_gather(buf):
        pltpu.make_async_copy(table_hbm.at[idxb.at[buf]], rows[buf],
                              gsem).wait()

    def transpose(buf):
        rbuf = rows[buf]
        sbuf = stag[buf]

        def step(bt8, carry):
            for kb in range(8):
                i0 = bt8 * 128 + kb * 16
                base = bt8 * 1024 + kb * 16
                row_idx = riota + i0
                vs = [plsc.load_gather(rbuf, [row_idx, cols[s]])
                      for s in range(DIM)]
                for s in range(DIM):
                    plsc.store_scatter(sbuf, [woffs[s] + base], vs[s])
            return carry

        lax.fori_loop(0, 8, step, 0)

    def start_write(t, buf):
        l, sb = unit_lsb(t)
        for dt in (0, 1):
            pltpu.async_copy(stag[buf].at[pl.ds(dt * HALF, HALF)],
                             out_hbm.at[l, dt, pl.ds(sb * HALF, HALF)], wsem)

    def wait_write(t, buf):
        l, sb = unit_lsb(t)
        for dt in (0, 1):
            pltpu.make_async_copy(stag[buf].at[pl.ds(dt * HALF, HALF)],
                                  out_hbm.at[l, dt, pl.ds(sb * HALF, HALF)],
                                  wsem).wait()

    load_add(0, 0)
    start_gather(0)

    def outer(tt, carry):
        for b in (0, 1):
            t = tt * 2 + b
            bnext = 1 - b

            @pl.when(t + 1 < UPT)
            def _():
                load_add(t + 1, bnext)

            wait_gather(b)

            @pl.when(t + 1 < UPT)
            def _():
                start_gather(bnext)

            @pl.when(t >= 2)
            def _():
                wait_write(t - 2, b)

            transpose(b)
            start_write(t, b)
        return carry

    lax.fori_loop(0, UPT // 2, outer, 0)
    wait_write(UPT - 2, 0)
    wait_write(UPT - 1, 1)


def kernel(action, table):
    idx2d = action.reshape(B, L).T + 1
    mesh = plsc.VectorSubcoreMesh(
        core_axis_name="c", subcore_axis_name="s", num_cores=NC,
        num_subcores=NS)
    out3 = pl.kernel(
        _body,
        out_type=jax.ShapeDtypeStruct((L, 2, BT * 8 * 128), jnp.float32),
        mesh=mesh,
        scratch_types=[
            pltpu.VMEM((2, SB), jnp.int32),
            pltpu.VMEM((SB, DIM), jnp.float32),
            pltpu.VMEM((SB, DIM), jnp.float32),
            pltpu.VMEM((2 * HALF,), jnp.float32),
            pltpu.VMEM((2 * HALF,), jnp.float32),
            pltpu.SemaphoreType.DMA,
            pltpu.SemaphoreType.DMA,
        ],
        compiler_params=pltpu.CompilerParams(use_tc_tiling_on_sc=False,
                                             needs_layout_passes=False),
    )(idx2d, table)
    out6 = out3.reshape(L, 2, BT, 8, 128)
    return out6.transpose(2, 4, 0, 1, 3).reshape(B, L, DIM)
